# baseline (speedup 1.0000x reference)
import functools

import numpy as np

import jax
import jax.numpy as jnp
from jax import lax
from jax.experimental import pallas as pl
from jax.experimental.pallas import tpu as pltpu
from jax.experimental.pallas import tpu_sc as plsc

_NB = 16384
_F = 26
_D = 64
_NC = 2
_NS = 16
_NW = _NC * _NS
_BC = 256
_NCH = _NB // _BC
_NJOB = _F * _NCH
_JPW = _NJOB // _NW


_COLC_NP = np.zeros((16, 4, 16), np.int32)
_STC_NP = np.zeros((16, 4, 16), np.int32)
for _m in range(16):
  for _k in range(4):
    for _l in range(16):
      _w = (_m + _l) & 15
      _COLC_NP[_m, _k, _l] = _k * 16 + _w
      _STC_NP[_m, _k, _l] = (2 * _k + _w // 8) * 2048 + (_w % 8) * 128 + _l
_COLC_2D = _COLC_NP.reshape(64, 16)
_STC_2D = _STC_NP.reshape(64, 16)


def _make_gather():
  mesh = plsc.VectorSubcoreMesh(core_axis_name="c", subcore_axis_name="s")

  @functools.partial(
      pl.kernel,
      out_type=jax.ShapeDtypeStruct((_F * 1048576,), jnp.float32),
      mesh=mesh,
      compiler_params=pltpu.CompilerParams(
          use_tc_tiling_on_sc=False, needs_layout_passes=False),
      scratch_types=[
          pltpu.VMEM((2, 128), jnp.int32),
          pltpu.VMEM((2, 128), jnp.int32),
          pltpu.VMEM((_BC, 128), jnp.float32),
          pltpu.VMEM((_BC, 128), jnp.float32),
          pltpu.VMEM((16384,), jnp.float32),
          pltpu.VMEM((16384,), jnp.float32),
          pltpu.VMEM((64, 16), jnp.int32),
          pltpu.VMEM((64, 16), jnp.int32),
          pltpu.SemaphoreType.DMA,
          pltpu.SemaphoreType.DMA,
          pltpu.SemaphoreType.DMA,
          pltpu.SemaphoreType.DMA,
      ],
  )
  def gather_kernel(idx_hbm, table_hbm, colc_hbm, stc_hbm, out_hbm, idx_v0,
                    idx_v1, rows_v0, rows_v1, tr_v0, tr_v1, colc_v, stc_v,
                    sem_g0, sem_g1, sem_s0, sem_s1):
    pltpu.sync_copy(colc_hbm, colc_v)
    pltpu.sync_copy(stc_hbm, stc_v)
    wid = lax.axis_index("s") * _NC + lax.axis_index("c")
    job0 = wid * _JPW
    idx_v = (idx_v0, idx_v1)
    rows = (rows_v0, rows_v1)
    tr = (tr_v0, tr_v1)
    sem_g = (sem_g0, sem_g1)
    sem_s = (sem_s0, sem_s1)

    def fire(j, b):
      f = j // _NCH
      c = lax.rem(j, _NCH)
      pltpu.sync_copy(idx_hbm.at[f, c], idx_v[b])
      for h in range(2):
        pltpu.async_copy(
            table_hbm.at[idx_v[b].at[h]],
            rows[b].at[pl.ds(h * 128, 128)],
            sem_g[b],
        )

    def wait_gathers(b):
      for h in range(2):
        pltpu.make_async_copy(
            table_hbm.at[idx_v[b].at[h]],
            rows[b].at[pl.ds(h * 128, 128)],
            sem_g[b],
        ).wait()

    def process(b):
      lanes = lax.iota(jnp.int32, 16)

      @plsc.parallel_loop(0, 16, unroll=2)
      def _trans(r16):
        rowvec = r16 * 16 + lanes
        sb = jnp.zeros((16,), jnp.int32) + ((r16 // 8) * 1024 + lax.rem(r16, 8) * 16)
        pend = []
        for kk in range(4):
          for m in range(16):
            ck = m * 4 + kk
            vals = plsc.load_gather(rows[b], [rowvec, colc_v[ck]])
            pend.append((ck, vals))
            if len(pend) >= 9:
              ck2, vals2 = pend.pop(0)
              plsc.store_scatter(tr[b], [sb + stc_v[ck2]], vals2)
        for ck2, vals2 in pend:
          plsc.store_scatter(tr[b], [sb + stc_v[ck2]], vals2)

    def fire_store(j, b):
      f = j // _NCH
      c = lax.rem(j, _NCH)
      for s0 in range(8):
        pltpu.async_copy(
            tr[b].at[pl.ds(s0 * 2048, 2048)],
            out_hbm.at[pl.ds(f * 1048576 + s0 * 131072 + c * 2048, 2048)],
            sem_s[b],
        )

    def wait_store(b):
      for s0 in range(8):
        pltpu.make_async_copy(
            tr[b].at[pl.ds(s0 * 2048, 2048)],
            out_hbm.at[pl.ds(s0 * 131072, 2048)],
            sem_s[b],
        ).wait()

    fire(job0, 0)

    @pl.loop(0, _JPW, step=2)
    def _outer(t0):
      for b in range(2):
        t = t0 + b
        nxt = t + 1

        @pl.when(nxt < _JPW)
        def _():
          @pl.when(nxt >= 2)
          def _():
            wait_store(1 - b)
          fire(job0 + nxt, 1 - b)

        wait_gathers(b)
        process(b)
        fire_store(job0 + t, b)

    wait_store(0)
    wait_store(1)

  return gather_kernel


_gather = _make_gather()


@jax.jit
def kernel(concept_indices, table):
  idx4d = concept_indices.T.reshape(_F, _NCH, 2, 128)
  table_p = jnp.pad(table, ((0, 0), (0, _D)))
  out1d = _gather(idx4d, table_p, jnp.asarray(_COLC_2D), jnp.asarray(_STC_2D))
  out5d = out1d.reshape(_F, 8, _NB // 128, 8, 128)
  return out5d.transpose(2, 4, 0, 1, 3).reshape(_NB, _F, _D)


# speedup vs baseline: 1.0161x; 1.0161x over previous
import functools

import numpy as np

import jax
import jax.numpy as jnp
from jax import lax
from jax.experimental import pallas as pl
from jax.experimental.pallas import tpu as pltpu
from jax.experimental.pallas import tpu_sc as plsc

_NB = 16384
_F = 26
_D = 64
_NC = 2
_NS = 16
_NW = _NC * _NS
_BC = 256
_NCH = _NB // _BC
_NJOB = _F * _NCH
_JPW = _NJOB // _NW


_COLC_NP = np.zeros((16, 4, 16), np.int32)
_STC_NP = np.zeros((16, 4, 16), np.int32)
for _m in range(16):
  for _k in range(4):
    for _l in range(16):
      _w = (_m + _l) & 15
      _COLC_NP[_m, _k, _l] = _k * 16 + _w
      _STC_NP[_m, _k, _l] = (2 * _k + _w // 8) * 2048 + (_w % 8) * 128 + _l
_COLC_2D = _COLC_NP.reshape(64, 16)
_STC_2D = _STC_NP.reshape(64, 16)


def _make_gather():
  mesh = plsc.VectorSubcoreMesh(core_axis_name="c", subcore_axis_name="s")

  @functools.partial(
      pl.kernel,
      out_type=jax.ShapeDtypeStruct((_F * 1048576,), jnp.float32),
      mesh=mesh,
      compiler_params=pltpu.CompilerParams(
          use_tc_tiling_on_sc=False, needs_layout_passes=False),
      scratch_types=[
          pltpu.VMEM((2, 128), jnp.int32),
          pltpu.VMEM((2, 128), jnp.int32),
          pltpu.VMEM((_BC, 128), jnp.float32),
          pltpu.VMEM((_BC, 128), jnp.float32),
          pltpu.VMEM((16384,), jnp.float32),
          pltpu.VMEM((16384,), jnp.float32),
          pltpu.VMEM((64, 16), jnp.int32),
          pltpu.VMEM((64, 16), jnp.int32),
          pltpu.SemaphoreType.DMA,
          pltpu.SemaphoreType.DMA,
          pltpu.SemaphoreType.DMA,
          pltpu.SemaphoreType.DMA,
      ],
  )
  def gather_kernel(idx_hbm, table_hbm, colc_hbm, stc_hbm, out_hbm, idx_v0,
                    idx_v1, rows_v0, rows_v1, tr_v0, tr_v1, colc_v, stc_v,
                    sem_g0, sem_g1, sem_s0, sem_s1):
    pltpu.sync_copy(colc_hbm, colc_v)
    pltpu.sync_copy(stc_hbm, stc_v)
    wid = lax.axis_index("s") * _NC + lax.axis_index("c")
    job0 = wid * _JPW
    idx_v = (idx_v0, idx_v1)
    rows = (rows_v0, rows_v1)
    tr = (tr_v0, tr_v1)
    sem_g = (sem_g0, sem_g1)
    sem_s = (sem_s0, sem_s1)

    def fire(j, b):
      f = j // _NCH
      c = lax.rem(j, _NCH)
      pltpu.sync_copy(idx_hbm.at[f, c], idx_v[b])
      for h in range(2):
        pltpu.async_copy(
            table_hbm.at[idx_v[b].at[h]],
            rows[b].at[pl.ds(h * 128, 128)],
            sem_g[b],
        )

    def wait_gathers(b):
      for h in range(2):
        pltpu.make_async_copy(
            table_hbm.at[idx_v[b].at[h]],
            rows[b].at[pl.ds(h * 128, 128)],
            sem_g[b],
        ).wait()

    def process(b):
      lanes = lax.iota(jnp.int32, 16)

      @plsc.parallel_loop(0, 16, unroll=2)
      def _trans(r16):
        rowvec = r16 * 16 + lanes
        sb = jnp.zeros((16,), jnp.int32) + ((r16 // 8) * 1024 + lax.rem(r16, 8) * 16)
        pend = []
        for kk in range(4):
          for m in range(16):
            ck = m * 4 + kk
            vals = plsc.load_gather(rows[b], [rowvec, colc_v[ck]])
            pend.append((ck, vals))
            if len(pend) >= 4:
              ck2, vals2 = pend.pop(0)
              plsc.store_scatter(tr[b], [sb + stc_v[ck2]], vals2)
        for ck2, vals2 in pend:
          plsc.store_scatter(tr[b], [sb + stc_v[ck2]], vals2)

    def fire_store(j, b):
      f = j // _NCH
      c = lax.rem(j, _NCH)
      for s0 in range(8):
        pltpu.async_copy(
            tr[b].at[pl.ds(s0 * 2048, 2048)],
            out_hbm.at[pl.ds(f * 1048576 + s0 * 131072 + c * 2048, 2048)],
            sem_s[b],
        )

    def wait_store(b):
      for s0 in range(8):
        pltpu.make_async_copy(
            tr[b].at[pl.ds(s0 * 2048, 2048)],
            out_hbm.at[pl.ds(s0 * 131072, 2048)],
            sem_s[b],
        ).wait()

    fire(job0, 0)

    @pl.loop(0, _JPW, step=2)
    def _outer(t0):
      for b in range(2):
        t = t0 + b
        nxt = t + 1

        @pl.when(nxt < _JPW)
        def _():
          @pl.when(nxt >= 2)
          def _():
            wait_store(1 - b)
          fire(job0 + nxt, 1 - b)

        wait_gathers(b)
        process(b)
        fire_store(job0 + t, b)

    wait_store(0)
    wait_store(1)

  return gather_kernel


_gather = _make_gather()


@jax.jit
def kernel(concept_indices, table):
  idx4d = concept_indices.T.reshape(_F, _NCH, 2, 128)
  table_p = jnp.pad(table, ((0, 0), (0, _D)))
  out1d = _gather(idx4d, table_p, jnp.asarray(_COLC_2D), jnp.asarray(_STC_2D))
  out5d = out1d.reshape(_F, 8, _NB // 128, 8, 128)
  return out5d.transpose(2, 4, 0, 1, 3).reshape(_NB, _F, _D)


# speedup vs baseline: 1.0438x; 1.0273x over previous
import functools

import numpy as np

import jax
import jax.numpy as jnp
from jax import lax
from jax.experimental import pallas as pl
from jax.experimental.pallas import tpu as pltpu
from jax.experimental.pallas import tpu_sc as plsc

_NB = 16384
_F = 26
_D = 64
_NC = 2
_NS = 16
_NW = _NC * _NS
_BC = 256
_NCH = _NB // _BC
_NJOB = _F * _NCH
_JPW = _NJOB // _NW


_COLC_NP = np.zeros((16, 4, 16), np.int32)
_STC_NP = np.zeros((16, 4, 16), np.int32)
for _m in range(16):
  for _k in range(4):
    for _l in range(16):
      _w = (_m + _l) & 15
      _COLC_NP[_m, _k, _l] = _k * 16 + _w
      _STC_NP[_m, _k, _l] = (2 * _k + _w // 8) * 2048 + (_w % 8) * 128 + _l
_COLC_2D = _COLC_NP.reshape(64, 16)
_STC_2D = _STC_NP.reshape(64, 16)


def _make_gather():
  mesh = plsc.VectorSubcoreMesh(core_axis_name="c", subcore_axis_name="s")

  @functools.partial(
      pl.kernel,
      out_type=jax.ShapeDtypeStruct((_F * 1048576,), jnp.float32),
      mesh=mesh,
      compiler_params=pltpu.CompilerParams(
          use_tc_tiling_on_sc=False, needs_layout_passes=False),
      scratch_types=[
          pltpu.VMEM((2, 128), jnp.int32),
          pltpu.VMEM((2, 128), jnp.int32),
          pltpu.VMEM((2, 128), jnp.int32),
          pltpu.VMEM((2, 128), jnp.int32),
          pltpu.VMEM((_BC, 128), jnp.float32),
          pltpu.VMEM((_BC, 128), jnp.float32),
          pltpu.VMEM((16384,), jnp.float32),
          pltpu.VMEM((16384,), jnp.float32),
          pltpu.VMEM((64, 16), jnp.int32),
          pltpu.VMEM((64, 16), jnp.int32),
          pltpu.SemaphoreType.DMA,
          pltpu.SemaphoreType.DMA,
          pltpu.SemaphoreType.DMA,
          pltpu.SemaphoreType.DMA,
          pltpu.SemaphoreType.DMA,
          pltpu.SemaphoreType.DMA,
          pltpu.SemaphoreType.DMA,
          pltpu.SemaphoreType.DMA,
      ],
  )
  def gather_kernel(idx_hbm, table_hbm, colc_hbm, stc_hbm, out_hbm, idx_v0,
                    idx_v1, idx_v2, idx_v3, rows_v0, rows_v1, tr_v0, tr_v1,
                    colc_v, stc_v, sem_g0, sem_g1, sem_s0, sem_s1, sem_i0,
                    sem_i1, sem_i2, sem_i3):
    pltpu.sync_copy(colc_hbm, colc_v)
    pltpu.sync_copy(stc_hbm, stc_v)
    wid = lax.axis_index("s") * _NC + lax.axis_index("c")
    job0 = wid * _JPW
    idx_v = (idx_v0, idx_v1, idx_v2, idx_v3)
    rows = (rows_v0, rows_v1)
    tr = (tr_v0, tr_v1)
    sem_g = (sem_g0, sem_g1)
    sem_s = (sem_s0, sem_s1)
    sem_i = (sem_i0, sem_i1, sem_i2, sem_i3)

    def fire_idx(j, q):
      f = j // _NCH
      c = lax.rem(j, _NCH)
      pltpu.async_copy(idx_hbm.at[f, c], idx_v[q], sem_i[q])

    def wait_idx(q):
      pltpu.make_async_copy(idx_hbm.at[0, 0], idx_v[q], sem_i[q]).wait()

    def fire_gathers(b, q):
      for h in range(2):
        pltpu.async_copy(
            table_hbm.at[idx_v[q].at[h]],
            rows[b].at[pl.ds(h * 128, 128)],
            sem_g[b],
        )

    def wait_gathers(b):
      for h in range(2):
        pltpu.make_async_copy(
            table_hbm.at[idx_v[0].at[h]],
            rows[b].at[pl.ds(h * 128, 128)],
            sem_g[b],
        ).wait()

    def process(b):
      lanes = lax.iota(jnp.int32, 16)

      @plsc.parallel_loop(0, 16, unroll=2)
      def _trans(r16):
        rowvec = r16 * 16 + lanes
        sb = jnp.zeros((16,), jnp.int32) + ((r16 // 8) * 1024 + lax.rem(r16, 8) * 16)
        pend = []
        for kk in range(4):
          for m in range(16):
            ck = m * 4 + kk
            vals = plsc.load_gather(rows[b], [rowvec, colc_v[ck]])
            pend.append((ck, vals))
            if len(pend) >= 4:
              ck2, vals2 = pend.pop(0)
              plsc.store_scatter(tr[b], [sb + stc_v[ck2]], vals2)
        for ck2, vals2 in pend:
          plsc.store_scatter(tr[b], [sb + stc_v[ck2]], vals2)

    def fire_store(j, b):
      f = j // _NCH
      c = lax.rem(j, _NCH)
      for s0 in range(8):
        pltpu.async_copy(
            tr[b].at[pl.ds(s0 * 2048, 2048)],
            out_hbm.at[pl.ds(f * 1048576 + s0 * 131072 + c * 2048, 2048)],
            sem_s[b],
        )

    def wait_store(b):
      for s0 in range(8):
        pltpu.make_async_copy(
            tr[b].at[pl.ds(s0 * 2048, 2048)],
            out_hbm.at[pl.ds(s0 * 131072, 2048)],
            sem_s[b],
        ).wait()

    fire_idx(job0, 0)
    fire_idx(job0 + 1, 1)
    wait_idx(0)
    fire_gathers(0, 0)
    fire_idx(job0 + 2, 2)

    @pl.loop(0, _JPW, step=4)
    def _outer(t0):
      for u in range(4):
        b = u % 2
        t = t0 + u
        nxt = t + 1

        @pl.when(nxt < _JPW)
        def _():
          @pl.when(nxt >= 2)
          def _():
            wait_store(1 - b)
          wait_idx((u + 1) % 4)
          fire_gathers(1 - b, (u + 1) % 4)

          @pl.when(nxt + 2 < _JPW)
          def _():
            fire_idx(job0 + nxt + 2, (u + 3) % 4)

        wait_gathers(b)
        process(b)
        fire_store(job0 + t, b)

    wait_store(0)
    wait_store(1)

  return gather_kernel


_gather = _make_gather()


@jax.jit
def kernel(concept_indices, table):
  idx4d = concept_indices.T.reshape(_F, _NCH, 2, 128)
  table_p = jnp.pad(table, ((0, 0), (0, _D)))
  out1d = _gather(idx4d, table_p, jnp.asarray(_COLC_2D), jnp.asarray(_STC_2D))
  out5d = out1d.reshape(_F, 8, _NB // 128, 8, 128)
  return out5d.transpose(2, 4, 0, 1, 3).reshape(_NB, _F, _D)
